# Initial kernel scaffold; baseline (speedup 1.0000x reference)
import jax
import jax.numpy as jnp
from jax.experimental import pallas as pl


def kernel(x, edge_index, edge_attr, batch, W1, b1, W2, b2, W3, b3, Wl, bl):
    raise NotImplementedError("write your pallas kernel here")


# speedup vs baseline: 18.6339x; 18.6339x over previous
import functools

import jax
import jax.numpy as jnp
from jax import lax
from jax.experimental import pallas as pl
from jax.experimental.pallas import tpu as pltpu
from jax.experimental.pallas import tpu_sc as plsc

NC = 2
NS = 16
NW = NC * NS
L = 16

CH = 80
SCH = 40
RZ = 128


def _mesh():
  return plsc.VectorSubcoreMesh(
      core_axis_name="c", subcore_axis_name="s", num_cores=NC, num_subcores=NS)


_CPARAMS = pltpu.CompilerParams(use_tc_tiling_on_sc=False, needs_layout_passes=False)


def _bcast(v, j):
  idx = jnp.full((L,), j, jnp.int32)
  return jnp.take_along_axis(v, idx, axis=0, mode="promise_in_bounds")


def _rsqrt16(d):
  i = lax.bitcast_convert_type(d, jnp.int32)
  i = jnp.int32(0x5F3759DF) - lax.shift_right_logical(i, 1)
  y = lax.bitcast_convert_type(i, jnp.float32)
  for _ in range(3):
    y = y * (1.5 - 0.5 * d * y * y)
  return y


def _zero_vmem_2d(buf, rows):
  ncol = buf.shape[1] // L
  def body(i, _):
    for q in range(ncol):
      buf[i, pl.ds(q * L, L)] = jnp.zeros((L,), jnp.float32)
    return 0
  lax.fori_loop(0, rows, body, 0)


def _zero_vmem_1d(buf, n16):
  def body(i, _):
    buf[pl.ds(i * L, L)] = jnp.zeros((L,), jnp.float32)
    return 0
  lax.fori_loop(0, n16, body, 0)


def _deg_kernel(NP, n_rows):
  total_sch = n_rows // SCH
  kmax_sch = (total_sch + NW - 1) // NW
  nchunks = NP // RZ

  @functools.partial(
      pl.kernel, mesh=_mesh(), compiler_params=_CPARAMS,
      out_type=jax.ShapeDtypeStruct((NC * NP,), jnp.float32),
      scratch_types=[
          pltpu.VMEM((SCH, CH), jnp.int32),
          pltpu.VMEM((SCH, CH), jnp.float32),
          pltpu.VMEM((RZ,), jnp.float32),
          pltpu.VMEM_SHARED((NP,), jnp.float32),
      ])
  def k(dst_hbm, ew_hbm, out_hbm, didx, ewb, zbuf, acc):
    c = lax.axis_index("c")
    s = lax.axis_index("s")
    w = s * NC + c
    _zero_vmem_1d(zbuf, RZ // L)

    def zc(k_, _):
      cid = s + NS * k_
      @pl.when(cid < nchunks)
      def _():
        pltpu.sync_copy(zbuf, acc.at[pl.ds(cid * RZ, RZ)])
      return 0
    lax.fori_loop(0, (nchunks + NS - 1) // NS, zc, 0)
    plsc.subcore_barrier()

    def sc_body(i, _):
      sid = w + NW * i
      @pl.when(sid < total_sch)
      def _():
        r0 = sid * SCH
        pltpu.sync_copy(dst_hbm.at[pl.ds(r0, SCH), :], didx)
        pltpu.sync_copy(ew_hbm.at[pl.ds(r0, SCH), :], ewb)
        def ch_body(j, _):
          pltpu.sync_copy(ewb.at[j], acc.at[didx.at[j]], add=True)
          return 0
        lax.fori_loop(0, SCH, ch_body, 0)
      return 0
    lax.fori_loop(0, kmax_sch, sc_body, 0)
    plsc.subcore_barrier()

    def dr(k_, _):
      cid = s + NS * k_
      @pl.when(cid < nchunks)
      def _():
        pltpu.sync_copy(acc.at[pl.ds(cid * RZ, RZ)],
                        out_hbm.at[pl.ds(c * NP + cid * RZ, RZ)])
      return 0
    lax.fori_loop(0, (nchunks + NS - 1) // NS, dr, 0)

  return k


def _mp_kernel(NP, n_rows):
  total_sch = n_rows // SCH
  kmax_sch = (total_sch + NW - 1) // NW
  nchunks = NP // RZ
  NG = CH // L

  @functools.partial(
      pl.kernel, mesh=_mesh(), compiler_params=_CPARAMS,
      out_type=jax.ShapeDtypeStruct((NC, NP, L), jnp.float32),
      scratch_types=[
          pltpu.VMEM((SCH, CH), jnp.int32),
          pltpu.VMEM((SCH, CH), jnp.int32),
          pltpu.VMEM((SCH, CH), jnp.float32),
          pltpu.VMEM((CH, L), jnp.float32),
          pltpu.VMEM((RZ, L), jnp.float32),
          pltpu.VMEM_SHARED((NP, L), jnp.float32),
      ])
  def k(htab_hbm, src_hbm, dst_hbm, ew_hbm, out_hbm,
        sidx, didx, ewb, rows, zbuf, acc):
    c = lax.axis_index("c")
    s = lax.axis_index("s")
    w = s * NC + c
    _zero_vmem_2d(zbuf, RZ)

    def zc(k_, _):
      cid = s + NS * k_
      @pl.when(cid < nchunks)
      def _():
        pltpu.sync_copy(zbuf, acc.at[pl.ds(cid * RZ, RZ), :])
      return 0
    lax.fori_loop(0, (nchunks + NS - 1) // NS, zc, 0)
    plsc.subcore_barrier()

    def sc_body(i, _):
      sid = w + NW * i
      @pl.when(sid < total_sch)
      def _():
        r0 = sid * SCH
        pltpu.sync_copy(src_hbm.at[pl.ds(r0, SCH), :], sidx)
        pltpu.sync_copy(dst_hbm.at[pl.ds(r0, SCH), :], didx)
        pltpu.sync_copy(ew_hbm.at[pl.ds(r0, SCH), :], ewb)

        def ch_body(j, _):
          pltpu.sync_copy(htab_hbm.at[sidx.at[j]], rows)
          def grp(g, _):
            ewv = ewb[j, pl.ds(g * L, L)]
            for t in range(L):
              r = g * L + t
              rows[r, :] = rows[r, :] * _bcast(ewv, t)
            return 0
          lax.fori_loop(0, NG, grp, 0)
          pltpu.sync_copy(rows, acc.at[didx.at[j]], add=True)
          return 0
        lax.fori_loop(0, SCH, ch_body, 0)
      return 0
    lax.fori_loop(0, kmax_sch, sc_body, 0)
    plsc.subcore_barrier()

    def dr(k_, _):
      cid = s + NS * k_
      @pl.when(cid < nchunks)
      def _():
        pltpu.sync_copy(acc.at[pl.ds(cid * RZ, RZ), :],
                        out_hbm.at[c, pl.ds(cid * RZ, RZ), :])
      return 0
    lax.fori_loop(0, (nchunks + NS - 1) // NS, dr, 0)

  return k


def _matvec16(xrow, wt_rows):
  h = _bcast(xrow, 0) * wt_rows[0]
  for jj in range(1, L):
    h = h + _bcast(xrow, jj) * wt_rows[jj]
  return h


def _dense1_kernel(NP):
  nchunks = NP // RZ
  kmax = (nchunks + NW - 1) // NW

  @functools.partial(
      pl.kernel, mesh=_mesh(), compiler_params=_CPARAMS,
      out_type=(jax.ShapeDtypeStruct((NP,), jnp.float32),
                jax.ShapeDtypeStruct((NP, L), jnp.float32)),
      scratch_types=[
          pltpu.VMEM((RZ,), jnp.float32),
          pltpu.VMEM((RZ,), jnp.float32),
          pltpu.VMEM((RZ, L), jnp.float32),
          pltpu.VMEM((L, L), jnp.float32),
          pltpu.VMEM((RZ,), jnp.float32),
          pltpu.VMEM((RZ, L), jnp.float32),
      ])
  def k(degp_hbm, x_hbm, wt_hbm, dinv_hbm, ht_hbm,
        d0, d1, xb, wtb, dvb, htb):
    c = lax.axis_index("c")
    s = lax.axis_index("s")
    w = s * NC + c
    pltpu.sync_copy(wt_hbm, wtb)
    wt_rows = [wtb[jj, :] for jj in range(L)]

    def chunk(k_, _):
      cid = w + NW * k_
      @pl.when(cid < nchunks)
      def _():
        r0 = cid * RZ
        pltpu.sync_copy(degp_hbm.at[pl.ds(r0, RZ)], d0)
        pltpu.sync_copy(degp_hbm.at[pl.ds(NP + r0, RZ)], d1)
        pltpu.sync_copy(x_hbm.at[pl.ds(r0, RZ), :], xb)
        def grp(g, _):
          dv = d0[pl.ds(g * L, L)] + d1[pl.ds(g * L, L)] + 1.0
          y = _rsqrt16(dv)
          dvb[pl.ds(g * L, L)] = y
          for t in range(L):
            n = g * L + t
            h = _matvec16(xb[n, :], wt_rows)
            htb[n, :] = h * _bcast(y, t)
          return 0
        lax.fori_loop(0, RZ // L, grp, 0)
        pltpu.sync_copy(dvb, dinv_hbm.at[pl.ds(r0, RZ)])
        pltpu.sync_copy(htb, ht_hbm.at[pl.ds(r0, RZ), :])
      return 0
    lax.fori_loop(0, kmax, chunk, 0)

  return k


def _dense_mid_kernel(NP):
  nchunks = NP // RZ
  kmax = (nchunks + NW - 1) // NW

  @functools.partial(
      pl.kernel, mesh=_mesh(), compiler_params=_CPARAMS,
      out_type=jax.ShapeDtypeStruct((NP, L), jnp.float32),
      scratch_types=[
          pltpu.VMEM((RZ, L), jnp.float32),
          pltpu.VMEM((RZ, L), jnp.float32),
          pltpu.VMEM((RZ, L), jnp.float32),
          pltpu.VMEM((RZ,), jnp.float32),
          pltpu.VMEM((L, L), jnp.float32),
          pltpu.VMEM((L,), jnp.float32),
          pltpu.VMEM((RZ, L), jnp.float32),
      ])
  def k(accp_hbm, htp_hbm, dinv_hbm, b_hbm, wt_hbm, ht_hbm,
        a0, a1, hp, dvb, wtb, bb, htb):
    c = lax.axis_index("c")
    s = lax.axis_index("s")
    w = s * NC + c
    pltpu.sync_copy(wt_hbm, wtb)
    pltpu.sync_copy(b_hbm, bb)
    wt_rows = [wtb[jj, :] for jj in range(L)]
    bv = bb[...]

    def chunk(k_, _):
      cid = w + NW * k_
      @pl.when(cid < nchunks)
      def _():
        r0 = cid * RZ
        pltpu.sync_copy(accp_hbm.at[0, pl.ds(r0, RZ), :], a0)
        pltpu.sync_copy(accp_hbm.at[1, pl.ds(r0, RZ), :], a1)
        pltpu.sync_copy(htp_hbm.at[pl.ds(r0, RZ), :], hp)
        pltpu.sync_copy(dinv_hbm.at[pl.ds(r0, RZ)], dvb)
        def grp(g, _):
          y = dvb[pl.ds(g * L, L)]
          for t in range(L):
            n = g * L + t
            xr = (a0[n, :] + a1[n, :] + hp[n, :]) * _bcast(y, t) + bv
            xr = jnp.maximum(xr, 0.0)
            h = _matvec16(xr, wt_rows)
            htb[n, :] = h * _bcast(y, t)
          return 0
        lax.fori_loop(0, RZ // L, grp, 0)
        pltpu.sync_copy(htb, ht_hbm.at[pl.ds(r0, RZ), :])
      return 0
    lax.fori_loop(0, kmax, chunk, 0)

  return k


def _dense3_kernel(NP):
  nchunks = NP // RZ
  kmax = (nchunks + NW - 1) // NW

  @functools.partial(
      pl.kernel, mesh=_mesh(), compiler_params=_CPARAMS,
      out_type=(jax.ShapeDtypeStruct((NP, L), jnp.float32),
                jax.ShapeDtypeStruct((NP, L), jnp.float32)),
      scratch_types=[
          pltpu.VMEM((RZ, L), jnp.float32),
          pltpu.VMEM((RZ, L), jnp.float32),
          pltpu.VMEM((RZ, L), jnp.float32),
          pltpu.VMEM((RZ,), jnp.float32),
          pltpu.VMEM((L, L), jnp.float32),
          pltpu.VMEM((L, L), jnp.float32),
          pltpu.VMEM((L,), jnp.float32),
          pltpu.VMEM((RZ, L), jnp.float32),
          pltpu.VMEM((RZ, L), jnp.float32),
      ])
  def k(accp_hbm, htp_hbm, dinv_hbm, b_hbm, wta_hbm, wtb_hbm,
        hta_hbm, htb_hbm,
        a0, a1, hp, dvb, wta, wtb, bb, ha, hb):
    c = lax.axis_index("c")
    s = lax.axis_index("s")
    w = s * NC + c
    pltpu.sync_copy(wta_hbm, wta)
    pltpu.sync_copy(wtb_hbm, wtb)
    pltpu.sync_copy(b_hbm, bb)
    wta_rows = [wta[jj, :] for jj in range(L)]
    wtb_rows = [wtb[jj, :] for jj in range(L)]
    bv = bb[...]

    def chunk(k_, _):
      cid = w + NW * k_
      @pl.when(cid < nchunks)
      def _():
        r0 = cid * RZ
        pltpu.sync_copy(accp_hbm.at[0, pl.ds(r0, RZ), :], a0)
        pltpu.sync_copy(accp_hbm.at[1, pl.ds(r0, RZ), :], a1)
        pltpu.sync_copy(htp_hbm.at[pl.ds(r0, RZ), :], hp)
        pltpu.sync_copy(dinv_hbm.at[pl.ds(r0, RZ)], dvb)
        def grp(g, _):
          y = dvb[pl.ds(g * L, L)]
          for t in range(L):
            n = g * L + t
            yb = _bcast(y, t)
            xr = (a0[n, :] + a1[n, :] + hp[n, :]) * yb + bv
            xr = jnp.maximum(xr, 0.0)
            ha[n, :] = _matvec16(xr, wta_rows) * yb
            hb[n, :] = _matvec16(xr, wtb_rows) * yb
          return 0
        lax.fori_loop(0, RZ // L, grp, 0)
        pltpu.sync_copy(ha, hta_hbm.at[pl.ds(r0, RZ), :])
        pltpu.sync_copy(hb, htb_hbm.at[pl.ds(r0, RZ), :])
      return 0
    lax.fori_loop(0, kmax, chunk, 0)

  return k


def _pool_kernel(NP, PG):
  nchunks = NP // RZ
  kmax = (nchunks + NW - 1) // NW

  @functools.partial(
      pl.kernel, mesh=_mesh(), compiler_params=_CPARAMS,
      out_type=(jax.ShapeDtypeStruct((NW * (PG + 2) * 2 * L,), jnp.float32),
                jax.ShapeDtypeStruct((NW * 5 * L,), jnp.float32)),
      scratch_types=[
          pltpu.VMEM((RZ, L), jnp.float32),
          pltpu.VMEM((RZ, L), jnp.float32),
          pltpu.VMEM((RZ, L), jnp.float32),
          pltpu.VMEM((RZ, L), jnp.float32),
          pltpu.VMEM((RZ, L), jnp.float32),
          pltpu.VMEM((RZ, L), jnp.float32),
          pltpu.VMEM((RZ,), jnp.float32),
          pltpu.VMEM((RZ,), jnp.int32),
          pltpu.VMEM((L,), jnp.float32),
          pltpu.VMEM((L,), jnp.float32),
          pltpu.VMEM(((PG + 2) * 2 * L,), jnp.float32),
          pltpu.VMEM((5 * L,), jnp.float32),
      ])
  def k(accpa_hbm, accpb_hbm, hta_hbm, htb_hbm, dinv_hbm,
        b3a_hbm, b3b_hbm, batch_hbm, pools_hbm, cnts_hbm,
        aa0, aa1, ab0, ab1, hpa, hpb, dvb, btb, b3a, b3b, pool, cnt):
    c = lax.axis_index("c")
    s = lax.axis_index("s")
    w = s * NC + c
    pltpu.sync_copy(b3a_hbm, b3a)
    pltpu.sync_copy(b3b_hbm, b3b)
    bva = b3a[...]
    bvb = b3b[...]
    _zero_vmem_1d(pool, (PG + 2) * 2)
    _zero_vmem_1d(cnt, 5)
    ones = jnp.ones((L,), jnp.float32)
    iota = jnp.arange(L, dtype=jnp.int32)
    lane0 = iota == 0

    def chunk(k_, _):
      cid = w + NW * k_
      @pl.when(cid < nchunks)
      def _():
        r0 = cid * RZ
        pltpu.sync_copy(accpa_hbm.at[0, pl.ds(r0, RZ), :], aa0)
        pltpu.sync_copy(accpa_hbm.at[1, pl.ds(r0, RZ), :], aa1)
        pltpu.sync_copy(accpb_hbm.at[0, pl.ds(r0, RZ), :], ab0)
        pltpu.sync_copy(accpb_hbm.at[1, pl.ds(r0, RZ), :], ab1)
        pltpu.sync_copy(hta_hbm.at[pl.ds(r0, RZ), :], hpa)
        pltpu.sync_copy(htb_hbm.at[pl.ds(r0, RZ), :], hpb)
        pltpu.sync_copy(dinv_hbm.at[pl.ds(r0, RZ)], dvb)
        pltpu.sync_copy(batch_hbm.at[pl.ds(r0, RZ)], btb)
        def grp(g, _):
          y = dvb[pl.ds(g * L, L)]
          bt = btb[pl.ds(g * L, L)]
          for t in range(L):
            n = g * L + t
            yb = _bcast(y, t)
            gv = _bcast(bt, t)
            h3a = (aa0[n, :] + aa1[n, :] + hpa[n, :]) * yb + bva
            h3b = (ab0[n, :] + ab1[n, :] + hpb[n, :]) * yb + bvb
            base = gv * (2 * L) + iota
            plsc.addupdate_scatter(pool, [base], h3a)
            plsc.addupdate_scatter(pool, [base + L], h3b)
            plsc.addupdate_scatter(cnt, [gv], ones, mask=lane0)
          return 0
        lax.fori_loop(0, RZ // L, grp, 0)
      return 0
    lax.fori_loop(0, kmax, chunk, 0)
    psz = (PG + 2) * 2 * L
    pltpu.sync_copy(pool, pools_hbm.at[pl.ds(w * psz, psz)])
    pltpu.sync_copy(cnt, cnts_hbm.at[pl.ds(w * 5 * L, 5 * L)])

  return k


def _final_kernel(PG, G):
  @functools.partial(
      pl.kernel, mesh=_mesh(), compiler_params=_CPARAMS,
      out_type=jax.ShapeDtypeStruct((G, L), jnp.float32),
      scratch_types=[
          pltpu.VMEM((NW * (PG + 2) * 2 * L,), jnp.float32),
          pltpu.VMEM((NW * 5 * L,), jnp.float32),
          pltpu.VMEM((2 * L, L), jnp.float32),
          pltpu.VMEM((L,), jnp.float32),
          pltpu.VMEM((G, L), jnp.float32),
          pltpu.VMEM((5 * L,), jnp.float32),
      ])
  def k(pools_hbm, cnts_hbm, wl_hbm, bl_hbm, out_hbm,
        pv, cv, wlb, blb, ob, ctot):
    c = lax.axis_index("c")
    s = lax.axis_index("s")
    @pl.when(jnp.logical_and(c == 0, s == 0))
    def _():
      pltpu.sync_copy(pools_hbm, pv)
      pltpu.sync_copy(cnts_hbm, cv)
      pltpu.sync_copy(wl_hbm, wlb)
      pltpu.sync_copy(bl_hbm, blb)
      wl_rows = [wlb[jj, :] for jj in range(2 * L)]
      blv = blb[...]

      def csum(q, _):
        acc = cv[pl.ds(q * L, L)]
        def ct(t_, a):
          return a + cv[pl.ds(t_ * 5 * L + q * L, L)]
        acc = lax.fori_loop(1, NW, ct, acc)
        ctot[pl.ds(q * L, L)] = 1.0 / jnp.maximum(acc, 1.0)
        return 0
      lax.fori_loop(0, G // L, csum, 0)

      def graph(g, _):
        za = jnp.zeros((L,), jnp.float32)
        zb = jnp.zeros((L,), jnp.float32)
        psz = (PG + 2) * 2 * L
        def tsum(t_, ab):
          a, b = ab
          off = t_ * psz + g * 2 * L
          return (a + pv[pl.ds(off, L)], b + pv[pl.ds(off + L, L)])
        sa, sb = lax.fori_loop(0, NW, tsum, (za, zb))
        q = g // L
        minv_v = ctot[pl.ds(q * L, L)]
        mv = _bcast(minv_v, g - q * L)
        sa = sa * mv
        sb = sb * mv
        o = blv
        for t in range(L):
          o = o + _bcast(sa, t) * wl_rows[t]
          o = o + _bcast(sb, t) * wl_rows[L + t]
        ob[g, :] = o
        return 0
      lax.fori_loop(0, G, graph, 0)
      pltpu.sync_copy(ob, out_hbm)

  return k


def kernel(x, edge_index, edge_attr, batch, W1, b1, W2, b2, W3, b3, Wl, bl):
  N = x.shape[0]
  E = edge_index.shape[1]
  G = 64
  NP = ((N + 127) // 128) * 128
  f32 = jnp.float32

  src = edge_index[0].reshape(E // CH, CH)
  dst = edge_index[1].reshape(E // CH, CH)
  ew2 = edge_attr.reshape(E // CH, CH)
  n_rows = E // CH

  xpad = jnp.pad(x, ((0, NP - N), (0, L - x.shape[1])))
  batch_pad = jnp.concatenate(
      [batch, jnp.full((NP - N,), G, jnp.int32)]).astype(jnp.int32)

  def padw(wt):
    return jnp.pad(wt, ((0, L - wt.shape[0]), (0, L - wt.shape[1])))

  WT1 = padw(W1.T.astype(f32))
  WT2 = padw(W2.T.astype(f32))
  WT3A = jnp.pad(W3.T[:, :L], ((0, 0), (0, 0)))
  WT3B = W3.T[:, L:]
  b1p = jnp.pad(b1, (0, L - b1.shape[0]))
  b2p = b2
  b3A = b3[:L]
  b3B = b3[L:]
  WlT = jnp.pad(Wl.T, ((0, 0), (0, L - Wl.shape[0])))
  blp = jnp.pad(bl, (0, L - bl.shape[0]))

  degp = _deg_kernel(NP, n_rows)(dst, ew2)
  dinv, ht1 = _dense1_kernel(NP)(degp, xpad, WT1)
  acc1 = _mp_kernel(NP, n_rows)(ht1, src, dst, ew2)
  ht2 = _dense_mid_kernel(NP)(acc1, ht1, dinv, b1p, WT2)
  acc2 = _mp_kernel(NP, n_rows)(ht2, src, dst, ew2)
  ht3a, ht3b = _dense3_kernel(NP)(acc2, ht2, dinv, b2p, WT3A, WT3B)
  acc3a = _mp_kernel(NP, n_rows)(ht3a, src, dst, ew2)
  acc3b = _mp_kernel(NP, n_rows)(ht3b, src, dst, ew2)
  pools, cnts = _pool_kernel(NP, G)(
      acc3a, acc3b, ht3a, ht3b, dinv, b3A, b3B, batch_pad)
  out = _final_kernel(G, G)(pools, cnts, WlT, blp)
  return out[:, :2]


# speedup vs baseline: 31.3992x; 1.6851x over previous
import functools

import jax
import jax.numpy as jnp
from jax import lax
from jax.experimental import pallas as pl
from jax.experimental.pallas import tpu as pltpu
from jax.experimental.pallas import tpu_sc as plsc

NC = 2
NS = 16
NW = NC * NS
L = 16

CH = 80
SCH = 40
RZ = 128


def _mesh():
  return plsc.VectorSubcoreMesh(
      core_axis_name="c", subcore_axis_name="s", num_cores=NC, num_subcores=NS)


_CPARAMS = pltpu.CompilerParams(use_tc_tiling_on_sc=False, needs_layout_passes=False)


def _bcast(v, j):
  idx = jnp.full((L,), j, jnp.int32)
  return jnp.take_along_axis(v, idx, axis=0, mode="promise_in_bounds")


def _rsqrt16(d):
  i = lax.bitcast_convert_type(d, jnp.int32)
  i = jnp.int32(0x5F3759DF) - lax.shift_right_logical(i, 1)
  y = lax.bitcast_convert_type(i, jnp.float32)
  for _ in range(3):
    y = y * (1.5 - 0.5 * d * y * y)
  return y


def _zero_vmem_2d(buf, rows):
  ncol = buf.shape[1] // L
  def body(i, _):
    for q in range(ncol):
      buf[i, pl.ds(q * L, L)] = jnp.zeros((L,), jnp.float32)
    return 0
  lax.fori_loop(0, rows, body, 0)


def _zero_vmem_1d(buf, n16):
  def body(i, _):
    buf[pl.ds(i * L, L)] = jnp.zeros((L,), jnp.float32)
    return 0
  lax.fori_loop(0, n16, body, 0)


def _deg_kernel(NP, n_rows):
  total_sch = n_rows // SCH
  kmax_sch = (total_sch + NW - 1) // NW
  nchunks = NP // RZ

  @functools.partial(
      pl.kernel, mesh=_mesh(), compiler_params=_CPARAMS,
      out_type=jax.ShapeDtypeStruct((NC * NP,), jnp.float32),
      scratch_types=[
          pltpu.VMEM((SCH, CH), jnp.int32),
          pltpu.VMEM((SCH, CH), jnp.float32),
          pltpu.VMEM((RZ,), jnp.float32),
          pltpu.VMEM_SHARED((NP,), jnp.float32),
          pltpu.SemaphoreType.DMA,
      ])
  def k(dst_hbm, ew_hbm, out_hbm, didx, ewb, zbuf, acc, sem):
    c = lax.axis_index("c")
    s = lax.axis_index("s")
    w = s * NC + c
    _zero_vmem_1d(zbuf, RZ // L)

    def zc(k_, _):
      cid = s + NS * k_
      @pl.when(cid < nchunks)
      def _():
        pltpu.sync_copy(zbuf, acc.at[pl.ds(cid * RZ, RZ)])
      return 0
    lax.fori_loop(0, (nchunks + NS - 1) // NS, zc, 0)
    plsc.subcore_barrier()

    def sc_body(i, _):
      sid = w + NW * i
      @pl.when(sid < total_sch)
      def _():
        r0 = sid * SCH
        pltpu.sync_copy(dst_hbm.at[pl.ds(r0, SCH), :], didx)
        pltpu.sync_copy(ew_hbm.at[pl.ds(r0, SCH), :], ewb)
        def ch_body(j, _):
          pltpu.sync_copy(ewb.at[j], acc.at[didx.at[j]], add=True)
          return 0
        lax.fori_loop(0, SCH, ch_body, 0)
      return 0
    lax.fori_loop(0, kmax_sch, sc_body, 0)
    plsc.subcore_barrier()

    def dr(k_, _):
      cid = s + NS * k_
      @pl.when(cid < nchunks)
      def _():
        pltpu.sync_copy(acc.at[pl.ds(cid * RZ, RZ)],
                        out_hbm.at[pl.ds(c * NP + cid * RZ, RZ)])
      return 0
    lax.fori_loop(0, (nchunks + NS - 1) // NS, dr, 0)

  return k


def _mp_kernel(NP, n_rows):
  total_sch = n_rows // SCH
  kmax_sch = (total_sch + NW - 1) // NW
  nchunks = NP // RZ
  NG = CH // L

  NBUF = 2
  assert SCH % NBUF == 0

  @functools.partial(
      pl.kernel, mesh=_mesh(), compiler_params=_CPARAMS,
      out_type=jax.ShapeDtypeStruct((NC, NP, L), jnp.float32),
      scratch_types=[
          pltpu.VMEM((SCH, CH), jnp.int32),
          pltpu.VMEM((SCH, CH), jnp.int32),
          pltpu.VMEM((SCH, CH), jnp.float32),
          [pltpu.VMEM((CH, L), jnp.float32) for _ in range(NBUF)],
          [pltpu.SemaphoreType.DMA for _ in range(NBUF)],
          pltpu.VMEM((RZ, L), jnp.float32),
          pltpu.VMEM_SHARED((NP, L), jnp.float32),
      ])
  def k(htab_hbm, src_hbm, dst_hbm, ew_hbm, out_hbm,
        sidx, didx, ewb, rows, gsem, zbuf, acc):
    c = lax.axis_index("c")
    s = lax.axis_index("s")
    w = s * NC + c
    _zero_vmem_2d(zbuf, RZ)

    def zc(k_, _):
      cid = s + NS * k_
      @pl.when(cid < nchunks)
      def _():
        pltpu.sync_copy(zbuf, acc.at[pl.ds(cid * RZ, RZ), :])
      return 0
    lax.fori_loop(0, (nchunks + NS - 1) // NS, zc, 0)
    plsc.subcore_barrier()

    def gather(j, b):
      pltpu.async_copy(htab_hbm.at[sidx.at[j]], rows[b], gsem[b])

    def wait_g(j, b):
      pltpu.make_async_copy(htab_hbm.at[sidx.at[j]], rows[b], gsem[b]).wait()

    def scale(j, b):
      def grp(g, _):
        ewv = ewb[j, pl.ds(g * L, L)]
        for t in range(L):
          r = g * L + t
          rows[b][r, :] = rows[b][r, :] * _bcast(ewv, t)
        return 0
      lax.fori_loop(0, NG, grp, 0)

    def sc_body(i, _):
      sid = w + NW * i
      @pl.when(sid < total_sch)
      def _():
        r0 = sid * SCH
        pltpu.sync_copy(src_hbm.at[pl.ds(r0, SCH), :], sidx)
        pltpu.sync_copy(dst_hbm.at[pl.ds(r0, SCH), :], didx)
        pltpu.sync_copy(ew_hbm.at[pl.ds(r0, SCH), :], ewb)

        gather(0, 0)

        def pipe(p, _):
          for q in range(NBUF):
            jq = p * NBUF + q
            @pl.when(jq + 1 < SCH)
            def _():
              gather(jq + 1, 1 - q)
            wait_g(jq, q)
            scale(jq, q)
            pltpu.sync_copy(rows[q], acc.at[didx.at[jq]], add=True)
          return 0
        lax.fori_loop(0, SCH // NBUF, pipe, 0)
      return 0
    lax.fori_loop(0, kmax_sch, sc_body, 0)
    plsc.subcore_barrier()

    def dr(k_, _):
      cid = s + NS * k_
      @pl.when(cid < nchunks)
      def _():
        pltpu.sync_copy(acc.at[pl.ds(cid * RZ, RZ), :],
                        out_hbm.at[c, pl.ds(cid * RZ, RZ), :])
      return 0
    lax.fori_loop(0, (nchunks + NS - 1) // NS, dr, 0)

  return k


def _matvec16(xrow, wt_rows):
  h = _bcast(xrow, 0) * wt_rows[0]
  for jj in range(1, L):
    h = h + _bcast(xrow, jj) * wt_rows[jj]
  return h


def _dense1_kernel(NP):
  nchunks = NP // RZ
  kmax = (nchunks + NW - 1) // NW

  @functools.partial(
      pl.kernel, mesh=_mesh(), compiler_params=_CPARAMS,
      out_type=(jax.ShapeDtypeStruct((NP,), jnp.float32),
                jax.ShapeDtypeStruct((NP, L), jnp.float32)),
      scratch_types=[
          pltpu.VMEM((RZ,), jnp.float32),
          pltpu.VMEM((RZ,), jnp.float32),
          pltpu.VMEM((RZ, L), jnp.float32),
          pltpu.VMEM((L, L), jnp.float32),
          pltpu.VMEM((RZ,), jnp.float32),
          pltpu.VMEM((RZ, L), jnp.float32),
      ])
  def k(degp_hbm, x_hbm, wt_hbm, dinv_hbm, ht_hbm,
        d0, d1, xb, wtb, dvb, htb):
    c = lax.axis_index("c")
    s = lax.axis_index("s")
    w = s * NC + c
    pltpu.sync_copy(wt_hbm, wtb)
    wt_rows = [wtb[jj, :] for jj in range(L)]

    def chunk(k_, _):
      cid = w + NW * k_
      @pl.when(cid < nchunks)
      def _():
        r0 = cid * RZ
        pltpu.sync_copy(degp_hbm.at[pl.ds(r0, RZ)], d0)
        pltpu.sync_copy(degp_hbm.at[pl.ds(NP + r0, RZ)], d1)
        pltpu.sync_copy(x_hbm.at[pl.ds(r0, RZ), :], xb)
        def grp(g, _):
          dv = d0[pl.ds(g * L, L)] + d1[pl.ds(g * L, L)] + 1.0
          y = _rsqrt16(dv)
          dvb[pl.ds(g * L, L)] = y
          for t in range(L):
            n = g * L + t
            h = _matvec16(xb[n, :], wt_rows)
            htb[n, :] = h * _bcast(y, t)
          return 0
        lax.fori_loop(0, RZ // L, grp, 0)
        pltpu.sync_copy(dvb, dinv_hbm.at[pl.ds(r0, RZ)])
        pltpu.sync_copy(htb, ht_hbm.at[pl.ds(r0, RZ), :])
      return 0
    lax.fori_loop(0, kmax, chunk, 0)

  return k


def _dense_mid_kernel(NP):
  nchunks = NP // RZ
  kmax = (nchunks + NW - 1) // NW

  @functools.partial(
      pl.kernel, mesh=_mesh(), compiler_params=_CPARAMS,
      out_type=jax.ShapeDtypeStruct((NP, L), jnp.float32),
      scratch_types=[
          pltpu.VMEM((RZ, L), jnp.float32),
          pltpu.VMEM((RZ, L), jnp.float32),
          pltpu.VMEM((RZ, L), jnp.float32),
          pltpu.VMEM((RZ,), jnp.float32),
          pltpu.VMEM((L, L), jnp.float32),
          pltpu.VMEM((L,), jnp.float32),
          pltpu.VMEM((RZ, L), jnp.float32),
      ])
  def k(accp_hbm, htp_hbm, dinv_hbm, b_hbm, wt_hbm, ht_hbm,
        a0, a1, hp, dvb, wtb, bb, htb):
    c = lax.axis_index("c")
    s = lax.axis_index("s")
    w = s * NC + c
    pltpu.sync_copy(wt_hbm, wtb)
    pltpu.sync_copy(b_hbm, bb)
    wt_rows = [wtb[jj, :] for jj in range(L)]
    bv = bb[...]

    def chunk(k_, _):
      cid = w + NW * k_
      @pl.when(cid < nchunks)
      def _():
        r0 = cid * RZ
        pltpu.sync_copy(accp_hbm.at[0, pl.ds(r0, RZ), :], a0)
        pltpu.sync_copy(accp_hbm.at[1, pl.ds(r0, RZ), :], a1)
        pltpu.sync_copy(htp_hbm.at[pl.ds(r0, RZ), :], hp)
        pltpu.sync_copy(dinv_hbm.at[pl.ds(r0, RZ)], dvb)
        def grp(g, _):
          y = dvb[pl.ds(g * L, L)]
          for t in range(L):
            n = g * L + t
            xr = (a0[n, :] + a1[n, :] + hp[n, :]) * _bcast(y, t) + bv
            xr = jnp.maximum(xr, 0.0)
            h = _matvec16(xr, wt_rows)
            htb[n, :] = h * _bcast(y, t)
          return 0
        lax.fori_loop(0, RZ // L, grp, 0)
        pltpu.sync_copy(htb, ht_hbm.at[pl.ds(r0, RZ), :])
      return 0
    lax.fori_loop(0, kmax, chunk, 0)

  return k


def _dense3_kernel(NP):
  nchunks = NP // RZ
  kmax = (nchunks + NW - 1) // NW

  @functools.partial(
      pl.kernel, mesh=_mesh(), compiler_params=_CPARAMS,
      out_type=(jax.ShapeDtypeStruct((NP, L), jnp.float32),
                jax.ShapeDtypeStruct((NP, L), jnp.float32)),
      scratch_types=[
          pltpu.VMEM((RZ, L), jnp.float32),
          pltpu.VMEM((RZ, L), jnp.float32),
          pltpu.VMEM((RZ, L), jnp.float32),
          pltpu.VMEM((RZ,), jnp.float32),
          pltpu.VMEM((L, L), jnp.float32),
          pltpu.VMEM((L, L), jnp.float32),
          pltpu.VMEM((L,), jnp.float32),
          pltpu.VMEM((RZ, L), jnp.float32),
          pltpu.VMEM((RZ, L), jnp.float32),
      ])
  def k(accp_hbm, htp_hbm, dinv_hbm, b_hbm, wta_hbm, wtb_hbm,
        hta_hbm, htb_hbm,
        a0, a1, hp, dvb, wta, wtb, bb, ha, hb):
    c = lax.axis_index("c")
    s = lax.axis_index("s")
    w = s * NC + c
    pltpu.sync_copy(wta_hbm, wta)
    pltpu.sync_copy(wtb_hbm, wtb)
    pltpu.sync_copy(b_hbm, bb)
    wta_rows = [wta[jj, :] for jj in range(L)]
    wtb_rows = [wtb[jj, :] for jj in range(L)]
    bv = bb[...]

    def chunk(k_, _):
      cid = w + NW * k_
      @pl.when(cid < nchunks)
      def _():
        r0 = cid * RZ
        pltpu.sync_copy(accp_hbm.at[0, pl.ds(r0, RZ), :], a0)
        pltpu.sync_copy(accp_hbm.at[1, pl.ds(r0, RZ), :], a1)
        pltpu.sync_copy(htp_hbm.at[pl.ds(r0, RZ), :], hp)
        pltpu.sync_copy(dinv_hbm.at[pl.ds(r0, RZ)], dvb)
        def grp(g, _):
          y = dvb[pl.ds(g * L, L)]
          for t in range(L):
            n = g * L + t
            yb = _bcast(y, t)
            xr = (a0[n, :] + a1[n, :] + hp[n, :]) * yb + bv
            xr = jnp.maximum(xr, 0.0)
            ha[n, :] = _matvec16(xr, wta_rows) * yb
            hb[n, :] = _matvec16(xr, wtb_rows) * yb
          return 0
        lax.fori_loop(0, RZ // L, grp, 0)
        pltpu.sync_copy(ha, hta_hbm.at[pl.ds(r0, RZ), :])
        pltpu.sync_copy(hb, htb_hbm.at[pl.ds(r0, RZ), :])
      return 0
    lax.fori_loop(0, kmax, chunk, 0)

  return k


def _pool_kernel(NP, PG):
  nchunks = NP // RZ
  kmax = (nchunks + NW - 1) // NW

  @functools.partial(
      pl.kernel, mesh=_mesh(), compiler_params=_CPARAMS,
      out_type=(jax.ShapeDtypeStruct((NW * (PG + 2) * 2 * L,), jnp.float32),
                jax.ShapeDtypeStruct((NW * 5 * L,), jnp.float32)),
      scratch_types=[
          pltpu.VMEM((RZ, L), jnp.float32),
          pltpu.VMEM((RZ, L), jnp.float32),
          pltpu.VMEM((RZ, L), jnp.float32),
          pltpu.VMEM((RZ, L), jnp.float32),
          pltpu.VMEM((RZ, L), jnp.float32),
          pltpu.VMEM((RZ, L), jnp.float32),
          pltpu.VMEM((RZ,), jnp.float32),
          pltpu.VMEM((RZ,), jnp.int32),
          pltpu.VMEM((L,), jnp.float32),
          pltpu.VMEM((L,), jnp.float32),
          pltpu.VMEM(((PG + 2) * 2 * L,), jnp.float32),
          pltpu.VMEM((5 * L,), jnp.float32),
      ])
  def k(accpa_hbm, accpb_hbm, hta_hbm, htb_hbm, dinv_hbm,
        b3a_hbm, b3b_hbm, batch_hbm, pools_hbm, cnts_hbm,
        aa0, aa1, ab0, ab1, hpa, hpb, dvb, btb, b3a, b3b, pool, cnt):
    c = lax.axis_index("c")
    s = lax.axis_index("s")
    w = s * NC + c
    pltpu.sync_copy(b3a_hbm, b3a)
    pltpu.sync_copy(b3b_hbm, b3b)
    bva = b3a[...]
    bvb = b3b[...]
    _zero_vmem_1d(pool, (PG + 2) * 2)
    _zero_vmem_1d(cnt, 5)
    ones = jnp.ones((L,), jnp.float32)
    iota = jnp.arange(L, dtype=jnp.int32)
    lane0 = iota == 0

    def chunk(k_, _):
      cid = w + NW * k_
      @pl.when(cid < nchunks)
      def _():
        r0 = cid * RZ
        pltpu.sync_copy(accpa_hbm.at[0, pl.ds(r0, RZ), :], aa0)
        pltpu.sync_copy(accpa_hbm.at[1, pl.ds(r0, RZ), :], aa1)
        pltpu.sync_copy(accpb_hbm.at[0, pl.ds(r0, RZ), :], ab0)
        pltpu.sync_copy(accpb_hbm.at[1, pl.ds(r0, RZ), :], ab1)
        pltpu.sync_copy(hta_hbm.at[pl.ds(r0, RZ), :], hpa)
        pltpu.sync_copy(htb_hbm.at[pl.ds(r0, RZ), :], hpb)
        pltpu.sync_copy(dinv_hbm.at[pl.ds(r0, RZ)], dvb)
        pltpu.sync_copy(batch_hbm.at[pl.ds(r0, RZ)], btb)
        def grp(g, _):
          y = dvb[pl.ds(g * L, L)]
          bt = btb[pl.ds(g * L, L)]
          for t in range(L):
            n = g * L + t
            yb = _bcast(y, t)
            gv = _bcast(bt, t)
            h3a = (aa0[n, :] + aa1[n, :] + hpa[n, :]) * yb + bva
            h3b = (ab0[n, :] + ab1[n, :] + hpb[n, :]) * yb + bvb
            base = gv * (2 * L) + iota
            plsc.addupdate_scatter(pool, [base], h3a)
            plsc.addupdate_scatter(pool, [base + L], h3b)
            plsc.addupdate_scatter(cnt, [gv], ones, mask=lane0)
          return 0
        lax.fori_loop(0, RZ // L, grp, 0)
      return 0
    lax.fori_loop(0, kmax, chunk, 0)
    psz = (PG + 2) * 2 * L
    pltpu.sync_copy(pool, pools_hbm.at[pl.ds(w * psz, psz)])
    pltpu.sync_copy(cnt, cnts_hbm.at[pl.ds(w * 5 * L, 5 * L)])

  return k


def _final_kernel(PG, G):
  @functools.partial(
      pl.kernel, mesh=_mesh(), compiler_params=_CPARAMS,
      out_type=jax.ShapeDtypeStruct((G, L), jnp.float32),
      scratch_types=[
          pltpu.VMEM((NW * (PG + 2) * 2 * L,), jnp.float32),
          pltpu.VMEM((NW * 5 * L,), jnp.float32),
          pltpu.VMEM((2 * L, L), jnp.float32),
          pltpu.VMEM((L,), jnp.float32),
          pltpu.VMEM((G, L), jnp.float32),
          pltpu.VMEM((5 * L,), jnp.float32),
      ])
  def k(pools_hbm, cnts_hbm, wl_hbm, bl_hbm, out_hbm,
        pv, cv, wlb, blb, ob, ctot):
    c = lax.axis_index("c")
    s = lax.axis_index("s")
    @pl.when(jnp.logical_and(c == 0, s == 0))
    def _():
      pltpu.sync_copy(pools_hbm, pv)
      pltpu.sync_copy(cnts_hbm, cv)
      pltpu.sync_copy(wl_hbm, wlb)
      pltpu.sync_copy(bl_hbm, blb)
      wl_rows = [wlb[jj, :] for jj in range(2 * L)]
      blv = blb[...]

      def csum(q, _):
        acc = cv[pl.ds(q * L, L)]
        def ct(t_, a):
          return a + cv[pl.ds(t_ * 5 * L + q * L, L)]
        acc = lax.fori_loop(1, NW, ct, acc)
        ctot[pl.ds(q * L, L)] = 1.0 / jnp.maximum(acc, 1.0)
        return 0
      lax.fori_loop(0, G // L, csum, 0)

      def graph(g, _):
        za = jnp.zeros((L,), jnp.float32)
        zb = jnp.zeros((L,), jnp.float32)
        psz = (PG + 2) * 2 * L
        def tsum(t_, ab):
          a, b = ab
          off = t_ * psz + g * 2 * L
          return (a + pv[pl.ds(off, L)], b + pv[pl.ds(off + L, L)])
        sa, sb = lax.fori_loop(0, NW, tsum, (za, zb))
        q = g // L
        minv_v = ctot[pl.ds(q * L, L)]
        mv = _bcast(minv_v, g - q * L)
        sa = sa * mv
        sb = sb * mv
        o = blv
        for t in range(L):
          o = o + _bcast(sa, t) * wl_rows[t]
          o = o + _bcast(sb, t) * wl_rows[L + t]
        ob[g, :] = o
        return 0
      lax.fori_loop(0, G, graph, 0)
      pltpu.sync_copy(ob, out_hbm)

  return k


def kernel(x, edge_index, edge_attr, batch, W1, b1, W2, b2, W3, b3, Wl, bl):
  N = x.shape[0]
  E = edge_index.shape[1]
  G = 64
  NP = ((N + 127) // 128) * 128
  f32 = jnp.float32

  src = edge_index[0].reshape(E // CH, CH)
  dst = edge_index[1].reshape(E // CH, CH)
  ew2 = edge_attr.reshape(E // CH, CH)
  n_rows = E // CH

  xpad = jnp.pad(x, ((0, NP - N), (0, L - x.shape[1])))
  batch_pad = jnp.concatenate(
      [batch, jnp.full((NP - N,), G, jnp.int32)]).astype(jnp.int32)

  def padw(wt):
    return jnp.pad(wt, ((0, L - wt.shape[0]), (0, L - wt.shape[1])))

  WT1 = padw(W1.T.astype(f32))
  WT2 = padw(W2.T.astype(f32))
  WT3A = jnp.pad(W3.T[:, :L], ((0, 0), (0, 0)))
  WT3B = W3.T[:, L:]
  b1p = jnp.pad(b1, (0, L - b1.shape[0]))
  b2p = b2
  b3A = b3[:L]
  b3B = b3[L:]
  WlT = jnp.pad(Wl.T, ((0, 0), (0, L - Wl.shape[0])))
  blp = jnp.pad(bl, (0, L - bl.shape[0]))

  degp = _deg_kernel(NP, n_rows)(dst, ew2)
  dinv, ht1 = _dense1_kernel(NP)(degp, xpad, WT1)
  acc1 = _mp_kernel(NP, n_rows)(ht1, src, dst, ew2)
  ht2 = _dense_mid_kernel(NP)(acc1, ht1, dinv, b1p, WT2)
  acc2 = _mp_kernel(NP, n_rows)(ht2, src, dst, ew2)
  ht3a, ht3b = _dense3_kernel(NP)(acc2, ht2, dinv, b2p, WT3A, WT3B)
  acc3a = _mp_kernel(NP, n_rows)(ht3a, src, dst, ew2)
  acc3b = _mp_kernel(NP, n_rows)(ht3b, src, dst, ew2)
  pools, cnts = _pool_kernel(NP, G)(
      acc3a, acc3b, ht3a, ht3b, dinv, b3A, b3B, batch_pad)
  out = _final_kernel(G, G)(pools, cnts, WlT, blp)
  return out[:, :2]


# speedup vs baseline: 31.4224x; 1.0007x over previous
import functools

import jax
import jax.numpy as jnp
from jax import lax
from jax.experimental import pallas as pl
from jax.experimental.pallas import tpu as pltpu
from jax.experimental.pallas import tpu_sc as plsc

NC = 2
NS = 16
NW = NC * NS
L = 16

CH = 80
SCH = 40
RZ = 128


def _mesh():
  return plsc.VectorSubcoreMesh(
      core_axis_name="c", subcore_axis_name="s", num_cores=NC, num_subcores=NS)


_CPARAMS = pltpu.CompilerParams(use_tc_tiling_on_sc=False, needs_layout_passes=False)


def _bcast(v, j):
  idx = jnp.full((L,), j, jnp.int32)
  return jnp.take_along_axis(v, idx, axis=0, mode="promise_in_bounds")


def _rsqrt16(d):
  i = lax.bitcast_convert_type(d, jnp.int32)
  i = jnp.int32(0x5F3759DF) - lax.shift_right_logical(i, 1)
  y = lax.bitcast_convert_type(i, jnp.float32)
  for _ in range(3):
    y = y * (1.5 - 0.5 * d * y * y)
  return y


def _zero_vmem_2d(buf, rows):
  ncol = buf.shape[1] // L
  def body(i, _):
    for q in range(ncol):
      buf[i, pl.ds(q * L, L)] = jnp.zeros((L,), jnp.float32)
    return 0
  lax.fori_loop(0, rows, body, 0)


def _zero_vmem_1d(buf, n16):
  def body(i, _):
    buf[pl.ds(i * L, L)] = jnp.zeros((L,), jnp.float32)
    return 0
  lax.fori_loop(0, n16, body, 0)


def _deg_kernel(NP, n_rows):
  total_sch = n_rows // SCH
  kmax_sch = (total_sch + NW - 1) // NW
  nchunks = NP // RZ

  @functools.partial(
      pl.kernel, mesh=_mesh(), compiler_params=_CPARAMS,
      out_type=jax.ShapeDtypeStruct((NC * NP,), jnp.float32),
      scratch_types=[
          pltpu.VMEM((SCH, CH), jnp.int32),
          pltpu.VMEM((SCH, CH), jnp.float32),
          pltpu.VMEM((RZ,), jnp.float32),
          pltpu.VMEM_SHARED((NP,), jnp.float32),
          pltpu.SemaphoreType.DMA,
      ])
  def k(dst_hbm, ew_hbm, out_hbm, didx, ewb, zbuf, acc, sem):
    c = lax.axis_index("c")
    s = lax.axis_index("s")
    w = s * NC + c
    _zero_vmem_1d(zbuf, RZ // L)

    def zc(k_, _):
      cid = s + NS * k_
      @pl.when(cid < nchunks)
      def _():
        pltpu.sync_copy(zbuf, acc.at[pl.ds(cid * RZ, RZ)])
      return 0
    lax.fori_loop(0, (nchunks + NS - 1) // NS, zc, 0)
    plsc.subcore_barrier()

    def sc_body(i, _):
      sid = w + NW * i
      @pl.when(sid < total_sch)
      def _():
        r0 = sid * SCH
        pltpu.sync_copy(dst_hbm.at[pl.ds(r0, SCH), :], didx)
        pltpu.sync_copy(ew_hbm.at[pl.ds(r0, SCH), :], ewb)
        def ch_body(j, _):
          pltpu.sync_copy(ewb.at[j], acc.at[didx.at[j]], add=True)
          return 0
        lax.fori_loop(0, SCH, ch_body, 0)
      return 0
    lax.fori_loop(0, kmax_sch, sc_body, 0)
    plsc.subcore_barrier()

    def dr(k_, _):
      cid = s + NS * k_
      @pl.when(cid < nchunks)
      def _():
        pltpu.sync_copy(acc.at[pl.ds(cid * RZ, RZ)],
                        out_hbm.at[pl.ds(c * NP + cid * RZ, RZ)])
      return 0
    lax.fori_loop(0, (nchunks + NS - 1) // NS, dr, 0)

  return k


def _mp_kernel(NP, n_rows):
  total_sch = n_rows // SCH
  kmax_sch = (total_sch + NW - 1) // NW
  nchunks = NP // RZ
  NG = CH // L

  NBUF = 2
  assert SCH % NBUF == 0

  @functools.partial(
      pl.kernel, mesh=_mesh(), compiler_params=_CPARAMS,
      out_type=jax.ShapeDtypeStruct((NC, NP, L), jnp.float32),
      scratch_types=[
          pltpu.VMEM((SCH, CH), jnp.int32),
          pltpu.VMEM((SCH, CH), jnp.int32),
          pltpu.VMEM((SCH, CH), jnp.float32),
          [pltpu.VMEM((CH, L), jnp.float32) for _ in range(NBUF)],
          [pltpu.SemaphoreType.DMA for _ in range(NBUF)],
          [pltpu.SemaphoreType.DMA for _ in range(NBUF)],
          pltpu.VMEM((RZ, L), jnp.float32),
          pltpu.VMEM_SHARED((NP, L), jnp.float32),
      ])
  def k(htab_hbm, src_hbm, dst_hbm, ew_hbm, out_hbm,
        sidx, didx, ewb, rows, gsem, ssem, zbuf, acc):
    c = lax.axis_index("c")
    s = lax.axis_index("s")
    w = s * NC + c
    _zero_vmem_2d(zbuf, RZ)

    def zc(k_, _):
      cid = s + NS * k_
      @pl.when(cid < nchunks)
      def _():
        pltpu.sync_copy(zbuf, acc.at[pl.ds(cid * RZ, RZ), :])
      return 0
    lax.fori_loop(0, (nchunks + NS - 1) // NS, zc, 0)
    plsc.subcore_barrier()

    def gather(j, b):
      pltpu.async_copy(htab_hbm.at[sidx.at[j]], rows[b], gsem[b])

    def wait_g(j, b):
      pltpu.make_async_copy(htab_hbm.at[sidx.at[j]], rows[b], gsem[b]).wait()

    def scale(j, b):
      def grp(g, _):
        ewv = ewb[j, pl.ds(g * L, L)]
        for t in range(L):
          r = g * L + t
          rows[b][r, :] = rows[b][r, :] * _bcast(ewv, t)
        return 0
      lax.fori_loop(0, NG, grp, 0)

    def sc_body(i, _):
      sid = w + NW * i
      @pl.when(sid < total_sch)
      def _():
        r0 = sid * SCH
        pltpu.sync_copy(src_hbm.at[pl.ds(r0, SCH), :], sidx)
        pltpu.sync_copy(dst_hbm.at[pl.ds(r0, SCH), :], didx)
        pltpu.sync_copy(ew_hbm.at[pl.ds(r0, SCH), :], ewb)

        gather(0, 0)

        def pipe(p, _):
          for q in range(NBUF):
            jq = p * NBUF + q
            @pl.when(jq >= 1)
            def _():
              pltpu.make_async_copy(
                  rows[1 - q], acc.at[didx.at[jq - 1]], ssem[1 - q]).wait()
            @pl.when(jq + 1 < SCH)
            def _():
              gather(jq + 1, 1 - q)
            wait_g(jq, q)
            scale(jq, q)
            pltpu.async_copy(rows[q], acc.at[didx.at[jq]], ssem[q], add=True)
          return 0
        lax.fori_loop(0, SCH // NBUF, pipe, 0)
        pltpu.make_async_copy(
            rows[1], acc.at[didx.at[SCH - 1]], ssem[1]).wait()
      return 0
    lax.fori_loop(0, kmax_sch, sc_body, 0)
    plsc.subcore_barrier()

    def dr(k_, _):
      cid = s + NS * k_
      @pl.when(cid < nchunks)
      def _():
        pltpu.sync_copy(acc.at[pl.ds(cid * RZ, RZ), :],
                        out_hbm.at[c, pl.ds(cid * RZ, RZ), :])
      return 0
    lax.fori_loop(0, (nchunks + NS - 1) // NS, dr, 0)

  return k


def _matvec16(xrow, wt_rows):
  h = _bcast(xrow, 0) * wt_rows[0]
  for jj in range(1, L):
    h = h + _bcast(xrow, jj) * wt_rows[jj]
  return h


def _dense1_kernel(NP):
  nchunks = NP // RZ
  kmax = (nchunks + NW - 1) // NW

  @functools.partial(
      pl.kernel, mesh=_mesh(), compiler_params=_CPARAMS,
      out_type=(jax.ShapeDtypeStruct((NP,), jnp.float32),
                jax.ShapeDtypeStruct((NP, L), jnp.float32)),
      scratch_types=[
          pltpu.VMEM((RZ,), jnp.float32),
          pltpu.VMEM((RZ,), jnp.float32),
          pltpu.VMEM((RZ, L), jnp.float32),
          pltpu.VMEM((L, L), jnp.float32),
          pltpu.VMEM((RZ,), jnp.float32),
          pltpu.VMEM((RZ, L), jnp.float32),
      ])
  def k(degp_hbm, x_hbm, wt_hbm, dinv_hbm, ht_hbm,
        d0, d1, xb, wtb, dvb, htb):
    c = lax.axis_index("c")
    s = lax.axis_index("s")
    w = s * NC + c
    pltpu.sync_copy(wt_hbm, wtb)
    wt_rows = [wtb[jj, :] for jj in range(L)]

    def chunk(k_, _):
      cid = w + NW * k_
      @pl.when(cid < nchunks)
      def _():
        r0 = cid * RZ
        pltpu.sync_copy(degp_hbm.at[pl.ds(r0, RZ)], d0)
        pltpu.sync_copy(degp_hbm.at[pl.ds(NP + r0, RZ)], d1)
        pltpu.sync_copy(x_hbm.at[pl.ds(r0, RZ), :], xb)
        def grp(g, _):
          dv = d0[pl.ds(g * L, L)] + d1[pl.ds(g * L, L)] + 1.0
          y = _rsqrt16(dv)
          dvb[pl.ds(g * L, L)] = y
          for t in range(L):
            n = g * L + t
            h = _matvec16(xb[n, :], wt_rows)
            htb[n, :] = h * _bcast(y, t)
          return 0
        lax.fori_loop(0, RZ // L, grp, 0)
        pltpu.sync_copy(dvb, dinv_hbm.at[pl.ds(r0, RZ)])
        pltpu.sync_copy(htb, ht_hbm.at[pl.ds(r0, RZ), :])
      return 0
    lax.fori_loop(0, kmax, chunk, 0)

  return k


def _dense_mid_kernel(NP):
  nchunks = NP // RZ
  kmax = (nchunks + NW - 1) // NW

  @functools.partial(
      pl.kernel, mesh=_mesh(), compiler_params=_CPARAMS,
      out_type=jax.ShapeDtypeStruct((NP, L), jnp.float32),
      scratch_types=[
          pltpu.VMEM((RZ, L), jnp.float32),
          pltpu.VMEM((RZ, L), jnp.float32),
          pltpu.VMEM((RZ, L), jnp.float32),
          pltpu.VMEM((RZ,), jnp.float32),
          pltpu.VMEM((L, L), jnp.float32),
          pltpu.VMEM((L,), jnp.float32),
          pltpu.VMEM((RZ, L), jnp.float32),
      ])
  def k(accp_hbm, htp_hbm, dinv_hbm, b_hbm, wt_hbm, ht_hbm,
        a0, a1, hp, dvb, wtb, bb, htb):
    c = lax.axis_index("c")
    s = lax.axis_index("s")
    w = s * NC + c
    pltpu.sync_copy(wt_hbm, wtb)
    pltpu.sync_copy(b_hbm, bb)
    wt_rows = [wtb[jj, :] for jj in range(L)]
    bv = bb[...]

    def chunk(k_, _):
      cid = w + NW * k_
      @pl.when(cid < nchunks)
      def _():
        r0 = cid * RZ
        pltpu.sync_copy(accp_hbm.at[0, pl.ds(r0, RZ), :], a0)
        pltpu.sync_copy(accp_hbm.at[1, pl.ds(r0, RZ), :], a1)
        pltpu.sync_copy(htp_hbm.at[pl.ds(r0, RZ), :], hp)
        pltpu.sync_copy(dinv_hbm.at[pl.ds(r0, RZ)], dvb)
        def grp(g, _):
          y = dvb[pl.ds(g * L, L)]
          for t in range(L):
            n = g * L + t
            xr = (a0[n, :] + a1[n, :] + hp[n, :]) * _bcast(y, t) + bv
            xr = jnp.maximum(xr, 0.0)
            h = _matvec16(xr, wt_rows)
            htb[n, :] = h * _bcast(y, t)
          return 0
        lax.fori_loop(0, RZ // L, grp, 0)
        pltpu.sync_copy(htb, ht_hbm.at[pl.ds(r0, RZ), :])
      return 0
    lax.fori_loop(0, kmax, chunk, 0)

  return k


def _dense3_kernel(NP):
  nchunks = NP // RZ
  kmax = (nchunks + NW - 1) // NW

  @functools.partial(
      pl.kernel, mesh=_mesh(), compiler_params=_CPARAMS,
      out_type=(jax.ShapeDtypeStruct((NP, L), jnp.float32),
                jax.ShapeDtypeStruct((NP, L), jnp.float32)),
      scratch_types=[
          pltpu.VMEM((RZ, L), jnp.float32),
          pltpu.VMEM((RZ, L), jnp.float32),
          pltpu.VMEM((RZ, L), jnp.float32),
          pltpu.VMEM((RZ,), jnp.float32),
          pltpu.VMEM((L, L), jnp.float32),
          pltpu.VMEM((L, L), jnp.float32),
          pltpu.VMEM((L,), jnp.float32),
          pltpu.VMEM((RZ, L), jnp.float32),
          pltpu.VMEM((RZ, L), jnp.float32),
      ])
  def k(accp_hbm, htp_hbm, dinv_hbm, b_hbm, wta_hbm, wtb_hbm,
        hta_hbm, htb_hbm,
        a0, a1, hp, dvb, wta, wtb, bb, ha, hb):
    c = lax.axis_index("c")
    s = lax.axis_index("s")
    w = s * NC + c
    pltpu.sync_copy(wta_hbm, wta)
    pltpu.sync_copy(wtb_hbm, wtb)
    pltpu.sync_copy(b_hbm, bb)
    wta_rows = [wta[jj, :] for jj in range(L)]
    wtb_rows = [wtb[jj, :] for jj in range(L)]
    bv = bb[...]

    def chunk(k_, _):
      cid = w + NW * k_
      @pl.when(cid < nchunks)
      def _():
        r0 = cid * RZ
        pltpu.sync_copy(accp_hbm.at[0, pl.ds(r0, RZ), :], a0)
        pltpu.sync_copy(accp_hbm.at[1, pl.ds(r0, RZ), :], a1)
        pltpu.sync_copy(htp_hbm.at[pl.ds(r0, RZ), :], hp)
        pltpu.sync_copy(dinv_hbm.at[pl.ds(r0, RZ)], dvb)
        def grp(g, _):
          y = dvb[pl.ds(g * L, L)]
          for t in range(L):
            n = g * L + t
            yb = _bcast(y, t)
            xr = (a0[n, :] + a1[n, :] + hp[n, :]) * yb + bv
            xr = jnp.maximum(xr, 0.0)
            ha[n, :] = _matvec16(xr, wta_rows) * yb
            hb[n, :] = _matvec16(xr, wtb_rows) * yb
          return 0
        lax.fori_loop(0, RZ // L, grp, 0)
        pltpu.sync_copy(ha, hta_hbm.at[pl.ds(r0, RZ), :])
        pltpu.sync_copy(hb, htb_hbm.at[pl.ds(r0, RZ), :])
      return 0
    lax.fori_loop(0, kmax, chunk, 0)

  return k


def _pool_kernel(NP, PG):
  nchunks = NP // RZ
  kmax = (nchunks + NW - 1) // NW

  @functools.partial(
      pl.kernel, mesh=_mesh(), compiler_params=_CPARAMS,
      out_type=(jax.ShapeDtypeStruct((NW * (PG + 2) * 2 * L,), jnp.float32),
                jax.ShapeDtypeStruct((NW * 5 * L,), jnp.float32)),
      scratch_types=[
          pltpu.VMEM((RZ, L), jnp.float32),
          pltpu.VMEM((RZ, L), jnp.float32),
          pltpu.VMEM((RZ, L), jnp.float32),
          pltpu.VMEM((RZ, L), jnp.float32),
          pltpu.VMEM((RZ, L), jnp.float32),
          pltpu.VMEM((RZ, L), jnp.float32),
          pltpu.VMEM((RZ,), jnp.float32),
          pltpu.VMEM((RZ,), jnp.int32),
          pltpu.VMEM((L,), jnp.float32),
          pltpu.VMEM((L,), jnp.float32),
          pltpu.VMEM(((PG + 2) * 2 * L,), jnp.float32),
          pltpu.VMEM((5 * L,), jnp.float32),
      ])
  def k(accpa_hbm, accpb_hbm, hta_hbm, htb_hbm, dinv_hbm,
        b3a_hbm, b3b_hbm, batch_hbm, pools_hbm, cnts_hbm,
        aa0, aa1, ab0, ab1, hpa, hpb, dvb, btb, b3a, b3b, pool, cnt):
    c = lax.axis_index("c")
    s = lax.axis_index("s")
    w = s * NC + c
    pltpu.sync_copy(b3a_hbm, b3a)
    pltpu.sync_copy(b3b_hbm, b3b)
    bva = b3a[...]
    bvb = b3b[...]
    _zero_vmem_1d(pool, (PG + 2) * 2)
    _zero_vmem_1d(cnt, 5)
    ones = jnp.ones((L,), jnp.float32)
    iota = jnp.arange(L, dtype=jnp.int32)
    lane0 = iota == 0

    def chunk(k_, _):
      cid = w + NW * k_
      @pl.when(cid < nchunks)
      def _():
        r0 = cid * RZ
        pltpu.sync_copy(accpa_hbm.at[0, pl.ds(r0, RZ), :], aa0)
        pltpu.sync_copy(accpa_hbm.at[1, pl.ds(r0, RZ), :], aa1)
        pltpu.sync_copy(accpb_hbm.at[0, pl.ds(r0, RZ), :], ab0)
        pltpu.sync_copy(accpb_hbm.at[1, pl.ds(r0, RZ), :], ab1)
        pltpu.sync_copy(hta_hbm.at[pl.ds(r0, RZ), :], hpa)
        pltpu.sync_copy(htb_hbm.at[pl.ds(r0, RZ), :], hpb)
        pltpu.sync_copy(dinv_hbm.at[pl.ds(r0, RZ)], dvb)
        pltpu.sync_copy(batch_hbm.at[pl.ds(r0, RZ)], btb)
        def grp(g, _):
          y = dvb[pl.ds(g * L, L)]
          bt = btb[pl.ds(g * L, L)]
          for t in range(L):
            n = g * L + t
            yb = _bcast(y, t)
            gv = _bcast(bt, t)
            h3a = (aa0[n, :] + aa1[n, :] + hpa[n, :]) * yb + bva
            h3b = (ab0[n, :] + ab1[n, :] + hpb[n, :]) * yb + bvb
            base = gv * (2 * L) + iota
            plsc.addupdate_scatter(pool, [base], h3a)
            plsc.addupdate_scatter(pool, [base + L], h3b)
            plsc.addupdate_scatter(cnt, [gv], ones, mask=lane0)
          return 0
        lax.fori_loop(0, RZ // L, grp, 0)
      return 0
    lax.fori_loop(0, kmax, chunk, 0)
    psz = (PG + 2) * 2 * L
    pltpu.sync_copy(pool, pools_hbm.at[pl.ds(w * psz, psz)])
    pltpu.sync_copy(cnt, cnts_hbm.at[pl.ds(w * 5 * L, 5 * L)])

  return k


def _final_kernel(PG, G):
  @functools.partial(
      pl.kernel, mesh=_mesh(), compiler_params=_CPARAMS,
      out_type=jax.ShapeDtypeStruct((G, L), jnp.float32),
      scratch_types=[
          pltpu.VMEM((NW * (PG + 2) * 2 * L,), jnp.float32),
          pltpu.VMEM((NW * 5 * L,), jnp.float32),
          pltpu.VMEM((2 * L, L), jnp.float32),
          pltpu.VMEM((L,), jnp.float32),
          pltpu.VMEM((G, L), jnp.float32),
          pltpu.VMEM((5 * L,), jnp.float32),
      ])
  def k(pools_hbm, cnts_hbm, wl_hbm, bl_hbm, out_hbm,
        pv, cv, wlb, blb, ob, ctot):
    c = lax.axis_index("c")
    s = lax.axis_index("s")
    @pl.when(jnp.logical_and(c == 0, s == 0))
    def _():
      pltpu.sync_copy(pools_hbm, pv)
      pltpu.sync_copy(cnts_hbm, cv)
      pltpu.sync_copy(wl_hbm, wlb)
      pltpu.sync_copy(bl_hbm, blb)
      wl_rows = [wlb[jj, :] for jj in range(2 * L)]
      blv = blb[...]

      def csum(q, _):
        acc = cv[pl.ds(q * L, L)]
        def ct(t_, a):
          return a + cv[pl.ds(t_ * 5 * L + q * L, L)]
        acc = lax.fori_loop(1, NW, ct, acc)
        ctot[pl.ds(q * L, L)] = 1.0 / jnp.maximum(acc, 1.0)
        return 0
      lax.fori_loop(0, G // L, csum, 0)

      def graph(g, _):
        za = jnp.zeros((L,), jnp.float32)
        zb = jnp.zeros((L,), jnp.float32)
        psz = (PG + 2) * 2 * L
        def tsum(t_, ab):
          a, b = ab
          off = t_ * psz + g * 2 * L
          return (a + pv[pl.ds(off, L)], b + pv[pl.ds(off + L, L)])
        sa, sb = lax.fori_loop(0, NW, tsum, (za, zb))
        q = g // L
        minv_v = ctot[pl.ds(q * L, L)]
        mv = _bcast(minv_v, g - q * L)
        sa = sa * mv
        sb = sb * mv
        o = blv
        for t in range(L):
          o = o + _bcast(sa, t) * wl_rows[t]
          o = o + _bcast(sb, t) * wl_rows[L + t]
        ob[g, :] = o
        return 0
      lax.fori_loop(0, G, graph, 0)
      pltpu.sync_copy(ob, out_hbm)

  return k


def kernel(x, edge_index, edge_attr, batch, W1, b1, W2, b2, W3, b3, Wl, bl):
  N = x.shape[0]
  E = edge_index.shape[1]
  G = 64
  NP = ((N + 127) // 128) * 128
  f32 = jnp.float32

  src = edge_index[0].reshape(E // CH, CH)
  dst = edge_index[1].reshape(E // CH, CH)
  ew2 = edge_attr.reshape(E // CH, CH)
  n_rows = E // CH

  xpad = jnp.pad(x, ((0, NP - N), (0, L - x.shape[1])))
  batch_pad = jnp.concatenate(
      [batch, jnp.full((NP - N,), G, jnp.int32)]).astype(jnp.int32)

  def padw(wt):
    return jnp.pad(wt, ((0, L - wt.shape[0]), (0, L - wt.shape[1])))

  WT1 = padw(W1.T.astype(f32))
  WT2 = padw(W2.T.astype(f32))
  WT3A = jnp.pad(W3.T[:, :L], ((0, 0), (0, 0)))
  WT3B = W3.T[:, L:]
  b1p = jnp.pad(b1, (0, L - b1.shape[0]))
  b2p = b2
  b3A = b3[:L]
  b3B = b3[L:]
  WlT = jnp.pad(Wl.T, ((0, 0), (0, L - Wl.shape[0])))
  blp = jnp.pad(bl, (0, L - bl.shape[0]))

  degp = _deg_kernel(NP, n_rows)(dst, ew2)
  dinv, ht1 = _dense1_kernel(NP)(degp, xpad, WT1)
  acc1 = _mp_kernel(NP, n_rows)(ht1, src, dst, ew2)
  ht2 = _dense_mid_kernel(NP)(acc1, ht1, dinv, b1p, WT2)
  acc2 = _mp_kernel(NP, n_rows)(ht2, src, dst, ew2)
  ht3a, ht3b = _dense3_kernel(NP)(acc2, ht2, dinv, b2p, WT3A, WT3B)
  acc3a = _mp_kernel(NP, n_rows)(ht3a, src, dst, ew2)
  acc3b = _mp_kernel(NP, n_rows)(ht3b, src, dst, ew2)
  pools, cnts = _pool_kernel(NP, G)(
      acc3a, acc3b, ht3a, ht3b, dinv, b3A, b3B, batch_pad)
  out = _final_kernel(G, G)(pools, cnts, WlT, blp)
  return out[:, :2]


# speedup vs baseline: 40.5635x; 1.2909x over previous
import functools

import jax
import jax.numpy as jnp
from jax import lax
from jax.experimental import pallas as pl
from jax.experimental.pallas import tpu as pltpu
from jax.experimental.pallas import tpu_sc as plsc

NC = 2
NS = 16
NW = NC * NS
L = 16

CH = 128
SCH = 40
RZ = 128


def _mesh():
  return plsc.VectorSubcoreMesh(
      core_axis_name="c", subcore_axis_name="s", num_cores=NC, num_subcores=NS)


_CPARAMS = pltpu.CompilerParams(use_tc_tiling_on_sc=False, needs_layout_passes=False)


def _bcast(v, j):
  idx = jnp.full((L,), j, jnp.int32)
  return jnp.take_along_axis(v, idx, axis=0, mode="promise_in_bounds")


def _rsqrt16(d):
  i = lax.bitcast_convert_type(d, jnp.int32)
  i = jnp.int32(0x5F3759DF) - lax.shift_right_logical(i, 1)
  y = lax.bitcast_convert_type(i, jnp.float32)
  for _ in range(3):
    y = y * (1.5 - 0.5 * d * y * y)
  return y


def _zero_vmem_2d(buf, rows):
  ncol = buf.shape[1] // L
  def body(i, _):
    for q in range(ncol):
      buf[i, pl.ds(q * L, L)] = jnp.zeros((L,), jnp.float32)
    return 0
  lax.fori_loop(0, rows, body, 0)


def _zero_vmem_1d(buf, n16):
  def body(i, _):
    buf[pl.ds(i * L, L)] = jnp.zeros((L,), jnp.float32)
    return 0
  lax.fori_loop(0, n16, body, 0)


def _deg_kernel(NP, n_rows):
  total_sch = n_rows // SCH
  kmax_sch = (total_sch + NW - 1) // NW
  nchunks = NP // RZ

  @functools.partial(
      pl.kernel, mesh=_mesh(), compiler_params=_CPARAMS,
      out_type=jax.ShapeDtypeStruct((NC * NP,), jnp.float32),
      scratch_types=[
          pltpu.VMEM((SCH, CH), jnp.int32),
          pltpu.VMEM((SCH, CH), jnp.float32),
          pltpu.VMEM((RZ,), jnp.float32),
          pltpu.VMEM_SHARED((NP,), jnp.float32),
          pltpu.SemaphoreType.DMA,
      ])
  def k(dst_hbm, ew_hbm, out_hbm, didx, ewb, zbuf, acc, sem):
    c = lax.axis_index("c")
    s = lax.axis_index("s")
    w = s * NC + c
    _zero_vmem_1d(zbuf, RZ // L)

    def zc(k_, _):
      cid = s + NS * k_
      @pl.when(cid < nchunks)
      def _():
        pltpu.sync_copy(zbuf, acc.at[pl.ds(cid * RZ, RZ)])
      return 0
    lax.fori_loop(0, (nchunks + NS - 1) // NS, zc, 0)
    plsc.subcore_barrier()

    def sc_body(i, _):
      sid = w + NW * i
      @pl.when(sid < total_sch)
      def _():
        r0 = sid * SCH
        pltpu.sync_copy(dst_hbm.at[pl.ds(r0, SCH), :], didx)
        pltpu.sync_copy(ew_hbm.at[pl.ds(r0, SCH), :], ewb)
        def ch_body(j, _):
          pltpu.sync_copy(ewb.at[j], acc.at[didx.at[j]], add=True)
          return 0
        lax.fori_loop(0, SCH, ch_body, 0)
      return 0
    lax.fori_loop(0, kmax_sch, sc_body, 0)
    plsc.subcore_barrier()

    def dr(k_, _):
      cid = s + NS * k_
      @pl.when(cid < nchunks)
      def _():
        pltpu.sync_copy(acc.at[pl.ds(cid * RZ, RZ)],
                        out_hbm.at[pl.ds(c * NP + cid * RZ, RZ)])
      return 0
    lax.fori_loop(0, (nchunks + NS - 1) // NS, dr, 0)

  return k


def _mp_kernel(NP, n_rows):
  total_sch = n_rows // SCH
  kmax_sch = (total_sch + NW - 1) // NW
  nchunks = NP // RZ
  NG = CH // L

  NBUF = 2
  assert SCH % NBUF == 0

  @functools.partial(
      pl.kernel, mesh=_mesh(), compiler_params=_CPARAMS,
      out_type=jax.ShapeDtypeStruct((NC, NP, L), jnp.float32),
      scratch_types=[
          pltpu.VMEM((SCH, CH), jnp.int32),
          pltpu.VMEM((SCH, CH), jnp.int32),
          pltpu.VMEM((SCH, CH), jnp.float32),
          [pltpu.VMEM((CH, L), jnp.float32) for _ in range(NBUF)],
          [pltpu.SemaphoreType.DMA for _ in range(NBUF)],
          [pltpu.SemaphoreType.DMA for _ in range(NBUF)],
          pltpu.VMEM((RZ, L), jnp.float32),
          pltpu.VMEM_SHARED((NP, L), jnp.float32),
      ])
  def k(htab_hbm, src_hbm, dst_hbm, ew_hbm, out_hbm,
        sidx, didx, ewb, rows, gsem, ssem, zbuf, acc):
    c = lax.axis_index("c")
    s = lax.axis_index("s")
    w = s * NC + c
    _zero_vmem_2d(zbuf, RZ)

    def zc(k_, _):
      cid = s + NS * k_
      @pl.when(cid < nchunks)
      def _():
        pltpu.sync_copy(zbuf, acc.at[pl.ds(cid * RZ, RZ), :])
      return 0
    lax.fori_loop(0, (nchunks + NS - 1) // NS, zc, 0)
    plsc.subcore_barrier()

    def gather(j, b):
      pltpu.async_copy(htab_hbm.at[sidx.at[j]], rows[b], gsem[b])

    def wait_g(j, b):
      pltpu.make_async_copy(htab_hbm.at[sidx.at[j]], rows[b], gsem[b]).wait()

    def scale(j, b):
      def grp(g, _):
        ewv = ewb[j, pl.ds(g * L, L)]
        for t in range(L):
          r = g * L + t
          rows[b][r, :] = rows[b][r, :] * _bcast(ewv, t)
        return 0
      lax.fori_loop(0, NG, grp, 0)

    def sc_body(i, _):
      sid = w + NW * i
      @pl.when(sid < total_sch)
      def _():
        r0 = sid * SCH
        pltpu.sync_copy(src_hbm.at[pl.ds(r0, SCH), :], sidx)
        pltpu.sync_copy(dst_hbm.at[pl.ds(r0, SCH), :], didx)
        pltpu.sync_copy(ew_hbm.at[pl.ds(r0, SCH), :], ewb)

        gather(0, 0)

        def pipe(p, _):
          for q in range(NBUF):
            jq = p * NBUF + q
            @pl.when(jq >= 1)
            def _():
              pltpu.make_async_copy(
                  rows[1 - q], acc.at[didx.at[jq - 1]], ssem[1 - q]).wait()
            @pl.when(jq + 1 < SCH)
            def _():
              gather(jq + 1, 1 - q)
            wait_g(jq, q)
            scale(jq, q)
            pltpu.async_copy(rows[q], acc.at[didx.at[jq]], ssem[q], add=True)
          return 0
        lax.fori_loop(0, SCH // NBUF, pipe, 0)
        pltpu.make_async_copy(
            rows[1], acc.at[didx.at[SCH - 1]], ssem[1]).wait()
      return 0
    lax.fori_loop(0, kmax_sch, sc_body, 0)
    plsc.subcore_barrier()

    def dr(k_, _):
      cid = s + NS * k_
      @pl.when(cid < nchunks)
      def _():
        pltpu.sync_copy(acc.at[pl.ds(cid * RZ, RZ), :],
                        out_hbm.at[c, pl.ds(cid * RZ, RZ), :])
      return 0
    lax.fori_loop(0, (nchunks + NS - 1) // NS, dr, 0)

  return k


def _matvec16(xrow, wt_rows):
  h = _bcast(xrow, 0) * wt_rows[0]
  for jj in range(1, L):
    h = h + _bcast(xrow, jj) * wt_rows[jj]
  return h


def _dense1_kernel(NP):
  nchunks = NP // RZ
  kmax = (nchunks + NW - 1) // NW

  @functools.partial(
      pl.kernel, mesh=_mesh(), compiler_params=_CPARAMS,
      out_type=(jax.ShapeDtypeStruct((NP,), jnp.float32),
                jax.ShapeDtypeStruct((NP, L), jnp.float32)),
      scratch_types=[
          pltpu.VMEM((RZ,), jnp.float32),
          pltpu.VMEM((RZ,), jnp.float32),
          pltpu.VMEM((RZ, L), jnp.float32),
          pltpu.VMEM((L, L), jnp.float32),
          pltpu.VMEM((RZ,), jnp.float32),
          pltpu.VMEM((RZ, L), jnp.float32),
      ])
  def k(degp_hbm, x_hbm, wt_hbm, dinv_hbm, ht_hbm,
        d0, d1, xb, wtb, dvb, htb):
    c = lax.axis_index("c")
    s = lax.axis_index("s")
    w = s * NC + c
    pltpu.sync_copy(wt_hbm, wtb)
    wt_rows = [wtb[jj, :] for jj in range(L)]

    def chunk(k_, _):
      cid = w + NW * k_
      @pl.when(cid < nchunks)
      def _():
        r0 = cid * RZ
        pltpu.sync_copy(degp_hbm.at[pl.ds(r0, RZ)], d0)
        pltpu.sync_copy(degp_hbm.at[pl.ds(NP + r0, RZ)], d1)
        pltpu.sync_copy(x_hbm.at[pl.ds(r0, RZ), :], xb)
        def grp(g, _):
          dv = d0[pl.ds(g * L, L)] + d1[pl.ds(g * L, L)] + 1.0
          y = _rsqrt16(dv)
          dvb[pl.ds(g * L, L)] = y
          for t in range(L):
            n = g * L + t
            h = _matvec16(xb[n, :], wt_rows)
            htb[n, :] = h * _bcast(y, t)
          return 0
        lax.fori_loop(0, RZ // L, grp, 0)
        pltpu.sync_copy(dvb, dinv_hbm.at[pl.ds(r0, RZ)])
        pltpu.sync_copy(htb, ht_hbm.at[pl.ds(r0, RZ), :])
      return 0
    lax.fori_loop(0, kmax, chunk, 0)

  return k


def _dense_mid_kernel(NP):
  nchunks = NP // RZ
  kmax = (nchunks + NW - 1) // NW

  @functools.partial(
      pl.kernel, mesh=_mesh(), compiler_params=_CPARAMS,
      out_type=jax.ShapeDtypeStruct((NP, L), jnp.float32),
      scratch_types=[
          pltpu.VMEM((RZ, L), jnp.float32),
          pltpu.VMEM((RZ, L), jnp.float32),
          pltpu.VMEM((RZ, L), jnp.float32),
          pltpu.VMEM((RZ,), jnp.float32),
          pltpu.VMEM((L, L), jnp.float32),
          pltpu.VMEM((L,), jnp.float32),
          pltpu.VMEM((RZ, L), jnp.float32),
      ])
  def k(accp_hbm, htp_hbm, dinv_hbm, b_hbm, wt_hbm, ht_hbm,
        a0, a1, hp, dvb, wtb, bb, htb):
    c = lax.axis_index("c")
    s = lax.axis_index("s")
    w = s * NC + c
    pltpu.sync_copy(wt_hbm, wtb)
    pltpu.sync_copy(b_hbm, bb)
    wt_rows = [wtb[jj, :] for jj in range(L)]
    bv = bb[...]

    def chunk(k_, _):
      cid = w + NW * k_
      @pl.when(cid < nchunks)
      def _():
        r0 = cid * RZ
        pltpu.sync_copy(accp_hbm.at[0, pl.ds(r0, RZ), :], a0)
        pltpu.sync_copy(accp_hbm.at[1, pl.ds(r0, RZ), :], a1)
        pltpu.sync_copy(htp_hbm.at[pl.ds(r0, RZ), :], hp)
        pltpu.sync_copy(dinv_hbm.at[pl.ds(r0, RZ)], dvb)
        def grp(g, _):
          y = dvb[pl.ds(g * L, L)]
          for t in range(L):
            n = g * L + t
            xr = (a0[n, :] + a1[n, :] + hp[n, :]) * _bcast(y, t) + bv
            xr = jnp.maximum(xr, 0.0)
            h = _matvec16(xr, wt_rows)
            htb[n, :] = h * _bcast(y, t)
          return 0
        lax.fori_loop(0, RZ // L, grp, 0)
        pltpu.sync_copy(htb, ht_hbm.at[pl.ds(r0, RZ), :])
      return 0
    lax.fori_loop(0, kmax, chunk, 0)

  return k


def _dense3_kernel(NP):
  nchunks = NP // RZ
  kmax = (nchunks + NW - 1) // NW

  @functools.partial(
      pl.kernel, mesh=_mesh(), compiler_params=_CPARAMS,
      out_type=(jax.ShapeDtypeStruct((NP, L), jnp.float32),
                jax.ShapeDtypeStruct((NP, L), jnp.float32)),
      scratch_types=[
          pltpu.VMEM((RZ, L), jnp.float32),
          pltpu.VMEM((RZ, L), jnp.float32),
          pltpu.VMEM((RZ, L), jnp.float32),
          pltpu.VMEM((RZ,), jnp.float32),
          pltpu.VMEM((L, L), jnp.float32),
          pltpu.VMEM((L, L), jnp.float32),
          pltpu.VMEM((L,), jnp.float32),
          pltpu.VMEM((RZ, L), jnp.float32),
          pltpu.VMEM((RZ, L), jnp.float32),
      ])
  def k(accp_hbm, htp_hbm, dinv_hbm, b_hbm, wta_hbm, wtb_hbm,
        hta_hbm, htb_hbm,
        a0, a1, hp, dvb, wta, wtb, bb, ha, hb):
    c = lax.axis_index("c")
    s = lax.axis_index("s")
    w = s * NC + c
    pltpu.sync_copy(wta_hbm, wta)
    pltpu.sync_copy(wtb_hbm, wtb)
    pltpu.sync_copy(b_hbm, bb)
    wta_rows = [wta[jj, :] for jj in range(L)]
    wtb_rows = [wtb[jj, :] for jj in range(L)]
    bv = bb[...]

    def chunk(k_, _):
      cid = w + NW * k_
      @pl.when(cid < nchunks)
      def _():
        r0 = cid * RZ
        pltpu.sync_copy(accp_hbm.at[0, pl.ds(r0, RZ), :], a0)
        pltpu.sync_copy(accp_hbm.at[1, pl.ds(r0, RZ), :], a1)
        pltpu.sync_copy(htp_hbm.at[pl.ds(r0, RZ), :], hp)
        pltpu.sync_copy(dinv_hbm.at[pl.ds(r0, RZ)], dvb)
        def grp(g, _):
          y = dvb[pl.ds(g * L, L)]
          for t in range(L):
            n = g * L + t
            yb = _bcast(y, t)
            xr = (a0[n, :] + a1[n, :] + hp[n, :]) * yb + bv
            xr = jnp.maximum(xr, 0.0)
            ha[n, :] = _matvec16(xr, wta_rows) * yb
            hb[n, :] = _matvec16(xr, wtb_rows) * yb
          return 0
        lax.fori_loop(0, RZ // L, grp, 0)
        pltpu.sync_copy(ha, hta_hbm.at[pl.ds(r0, RZ), :])
        pltpu.sync_copy(hb, htb_hbm.at[pl.ds(r0, RZ), :])
      return 0
    lax.fori_loop(0, kmax, chunk, 0)

  return k


def _pool_kernel(NP, PG):
  nchunks = NP // RZ
  kmax = (nchunks + NW - 1) // NW

  @functools.partial(
      pl.kernel, mesh=_mesh(), compiler_params=_CPARAMS,
      out_type=(jax.ShapeDtypeStruct((NW * 2 * (PG + 2) * 2 * L,), jnp.float32),
                jax.ShapeDtypeStruct((NW * 2 * 5 * L,), jnp.float32)),
      scratch_types=[
          pltpu.VMEM((RZ, L), jnp.float32),
          pltpu.VMEM((RZ, L), jnp.float32),
          pltpu.VMEM((RZ, L), jnp.float32),
          pltpu.VMEM((RZ, L), jnp.float32),
          pltpu.VMEM((RZ, L), jnp.float32),
          pltpu.VMEM((RZ, L), jnp.float32),
          pltpu.VMEM((RZ,), jnp.float32),
          pltpu.VMEM((RZ,), jnp.int32),
          pltpu.VMEM((L,), jnp.float32),
          pltpu.VMEM((L,), jnp.float32),
          pltpu.VMEM((2 * (PG + 2) * 2 * L,), jnp.float32),
          pltpu.VMEM((2 * 5 * L,), jnp.float32),
      ])
  def k(accpa_hbm, accpb_hbm, hta_hbm, htb_hbm, dinv_hbm,
        b3a_hbm, b3b_hbm, batch_hbm, pools_hbm, cnts_hbm,
        aa0, aa1, ab0, ab1, hpa, hpb, dvb, btb, b3a, b3b, pool, cnt):
    c = lax.axis_index("c")
    s = lax.axis_index("s")
    w = s * NC + c
    pltpu.sync_copy(b3a_hbm, b3a)
    pltpu.sync_copy(b3b_hbm, b3b)
    bva = b3a[...]
    bvb = b3b[...]
    _zero_vmem_1d(pool, 2 * (PG + 2) * 2)
    _zero_vmem_1d(cnt, 2 * 5)
    ones = jnp.ones((L,), jnp.float32)
    iota = jnp.arange(L, dtype=jnp.int32)
    lane0 = iota == 0

    def chunk(k_, _):
      cid = w + NW * k_
      @pl.when(cid < nchunks)
      def _():
        r0 = cid * RZ
        pltpu.sync_copy(accpa_hbm.at[0, pl.ds(r0, RZ), :], aa0)
        pltpu.sync_copy(accpa_hbm.at[1, pl.ds(r0, RZ), :], aa1)
        pltpu.sync_copy(accpb_hbm.at[0, pl.ds(r0, RZ), :], ab0)
        pltpu.sync_copy(accpb_hbm.at[1, pl.ds(r0, RZ), :], ab1)
        pltpu.sync_copy(hta_hbm.at[pl.ds(r0, RZ), :], hpa)
        pltpu.sync_copy(htb_hbm.at[pl.ds(r0, RZ), :], hpb)
        pltpu.sync_copy(dinv_hbm.at[pl.ds(r0, RZ)], dvb)
        pltpu.sync_copy(batch_hbm.at[pl.ds(r0, RZ)], btb)
        def grp(g, _):
          y = dvb[pl.ds(g * L, L)]
          bt = btb[pl.ds(g * L, L)]
          for t in range(L):
            n = g * L + t
            yb = _bcast(y, t)
            gv = _bcast(bt, t)
            h3a = (aa0[n, :] + aa1[n, :] + hpa[n, :]) * yb + bva
            h3b = (ab0[n, :] + ab1[n, :] + hpb[n, :]) * yb + bvb
            bank = t % 2
            base = bank * ((PG + 2) * 2 * L) + gv * (2 * L) + iota
            plsc.addupdate_scatter(pool, [base], h3a)
            plsc.addupdate_scatter(pool, [base + L], h3b)
            plsc.addupdate_scatter(cnt, [gv + bank * (5 * L)], ones,
                                   mask=lane0)
          return 0
        lax.fori_loop(0, RZ // L, grp, 0)
      return 0
    lax.fori_loop(0, kmax, chunk, 0)
    psz = 2 * (PG + 2) * 2 * L
    pltpu.sync_copy(pool, pools_hbm.at[pl.ds(w * psz, psz)])
    pltpu.sync_copy(cnt, cnts_hbm.at[pl.ds(w * 2 * 5 * L, 2 * 5 * L)])

  return k


def _final_kernel(PG, G):
  @functools.partial(
      pl.kernel, mesh=_mesh(), compiler_params=_CPARAMS,
      out_type=jax.ShapeDtypeStruct((G, L), jnp.float32),
      scratch_types=[
          pltpu.VMEM((NW * (PG + 2) * 2 * L,), jnp.float32),
          pltpu.VMEM((NW * 2 * 5 * L,), jnp.float32),
          pltpu.VMEM((2 * L, L), jnp.float32),
          pltpu.VMEM((L,), jnp.float32),
          pltpu.VMEM((G, L), jnp.float32),
          pltpu.VMEM((G, L), jnp.float32),
          pltpu.VMEM((G, L), jnp.float32),
          pltpu.VMEM((5 * L,), jnp.float32),
      ])
  def k(pools_hbm, cnts_hbm, wl_hbm, bl_hbm, out_hbm,
        pv, cv, wlb, blb, ob, sab, sbb, ctot):
    c = lax.axis_index("c")
    s = lax.axis_index("s")
    @pl.when(jnp.logical_and(c == 0, s == 0))
    def _():
      pltpu.sync_copy(cnts_hbm, cv)
      pltpu.sync_copy(wl_hbm, wlb)
      pltpu.sync_copy(bl_hbm, blb)
      wl_rows = [wlb[jj, :] for jj in range(2 * L)]
      blv = blb[...]
      _zero_vmem_2d(sab, G)
      _zero_vmem_2d(sbb, G)

      def csum(q, _):
        acc = cv[pl.ds(q * L, L)]
        def ct(t_, a):
          return a + cv[pl.ds(t_ * 5 * L + q * L, L)]
        acc = lax.fori_loop(1, NW * 2, ct, acc)
        ctot[pl.ds(q * L, L)] = 1.0 / jnp.maximum(acc, 1.0)
        return 0
      lax.fori_loop(0, G // L, csum, 0)

      psz = (PG + 2) * 2 * L
      half = NW * psz
      for h in range(2):
        pltpu.sync_copy(pools_hbm.at[pl.ds(h * half, half)], pv)
        def graph_acc(g, _):
          za = sab[g, :]
          zb = sbb[g, :]
          def tsum(t_, ab):
            a, b = ab
            off = t_ * psz + g * 2 * L
            return (a + pv[pl.ds(off, L)], b + pv[pl.ds(off + L, L)])
          sa, sb = lax.fori_loop(0, NW, tsum, (za, zb))
          sab[g, :] = sa
          sbb[g, :] = sb
          return 0
        lax.fori_loop(0, G, graph_acc, 0)

      def graph(g, _):
        q = g // L
        minv_v = ctot[pl.ds(q * L, L)]
        mv = _bcast(minv_v, g - q * L)
        sa = sab[g, :] * mv
        sb = sbb[g, :] * mv
        o = blv
        for t in range(L):
          o = o + _bcast(sa, t) * wl_rows[t]
          o = o + _bcast(sb, t) * wl_rows[L + t]
        ob[g, :] = o
        return 0
      lax.fori_loop(0, G, graph, 0)
      pltpu.sync_copy(ob, out_hbm)

  return k


def kernel(x, edge_index, edge_attr, batch, W1, b1, W2, b2, W3, b3, Wl, bl):
  N = x.shape[0]
  E = edge_index.shape[1]
  G = 64
  NP = ((N + 127) // 128) * 128
  f32 = jnp.float32

  src = edge_index[0].reshape(E // CH, CH)
  dst = edge_index[1].reshape(E // CH, CH)
  ew2 = edge_attr.reshape(E // CH, CH)
  n_rows = E // CH

  xpad = jnp.pad(x, ((0, NP - N), (0, L - x.shape[1])))
  batch_pad = jnp.concatenate(
      [batch, jnp.full((NP - N,), G, jnp.int32)]).astype(jnp.int32)

  def padw(wt):
    return jnp.pad(wt, ((0, L - wt.shape[0]), (0, L - wt.shape[1])))

  WT1 = padw(W1.T.astype(f32))
  WT2 = padw(W2.T.astype(f32))
  WT3A = jnp.pad(W3.T[:, :L], ((0, 0), (0, 0)))
  WT3B = W3.T[:, L:]
  b1p = jnp.pad(b1, (0, L - b1.shape[0]))
  b2p = b2
  b3A = b3[:L]
  b3B = b3[L:]
  WlT = jnp.pad(Wl.T, ((0, 0), (0, L - Wl.shape[0])))
  blp = jnp.pad(bl, (0, L - bl.shape[0]))

  degp = _deg_kernel(NP, n_rows)(dst, ew2)
  dinv, ht1 = _dense1_kernel(NP)(degp, xpad, WT1)
  acc1 = _mp_kernel(NP, n_rows)(ht1, src, dst, ew2)
  ht2 = _dense_mid_kernel(NP)(acc1, ht1, dinv, b1p, WT2)
  acc2 = _mp_kernel(NP, n_rows)(ht2, src, dst, ew2)
  ht3a, ht3b = _dense3_kernel(NP)(acc2, ht2, dinv, b2p, WT3A, WT3B)
  acc3a = _mp_kernel(NP, n_rows)(ht3a, src, dst, ew2)
  acc3b = _mp_kernel(NP, n_rows)(ht3b, src, dst, ew2)
  pools, cnts = _pool_kernel(NP, G)(
      acc3a, acc3b, ht3a, ht3b, dinv, b3A, b3B, batch_pad)
  out = _final_kernel(G, G)(pools, cnts, WlT, blp)
  return out[:, :2]
